# baseline (speedup 1.0000x reference)
import jax
import jax.numpy as jnp
from jax.experimental import pallas as pl
from jax.experimental.pallas import tpu as pltpu

N = 4096
BR = 256
NB = N // (2 * BR)
BC = 512
NC = N // BC


def _gcn_kernel(x_ref, adj_t_ref, adj_b_ref, w1_ref, b1_ref, w3_ref,
                b3_ref, wg_ref, bg_ref, out_ref, deg_ref, miss_ref,
                hwt_ref, vb_ref, cache_ref):
    i = pl.program_id(0)

    @pl.when(i == 0)
    def _init():
        deg_ref[...] = jnp.zeros_like(deg_ref)
        h = jax.nn.relu(
            jnp.dot(x_ref[...], w1_ref[...],
                    preferred_element_type=jnp.float32) + b1_ref[...])
        h = jax.nn.relu(
            jnp.dot(h, w3_ref[...],
                    preferred_element_type=jnp.float32) + b3_ref[...])
        hw = jnp.dot(h, wg_ref[...], preferred_element_type=jnp.float32)
        hwt_ref[...] = jnp.transpose(hw, (1, 0))

    ones_row = jnp.ones((1, BR), dtype=jnp.bfloat16)
    r_ids = jax.lax.broadcasted_iota(jnp.int32, (BR, BR), 0)
    c_ids = jax.lax.broadcasted_iota(jnp.int32, (BR, BR), 1)
    eye = (r_ids == c_ids).astype(jnp.float32)

    def _consume(blk_ref, row0):
        blk = blk_ref[...].astype(jnp.bfloat16)
        cache_ref[pl.ds(row0, BR), :] = blk
        deg_ref[...] += jnp.dot(ones_row, blk,
                                preferred_element_type=jnp.float32)
        sq = blk_ref[:, pl.ds(row0, BR)]
        diag_row = jnp.sum(sq * eye, axis=0, keepdims=True)
        miss_ref[:, pl.ds(row0, BR)] = jnp.where(diag_row > 0, 0.0, 1.0)

    _consume(adj_t_ref, i * BR)
    _consume(adj_b_ref, (i + NB) * BR)

    @pl.when(i == NB - 1)
    def _finalize():
        miss_row = miss_ref[...]
        deg = deg_ref[...] + miss_row
        dinv_row = jax.lax.rsqrt(jnp.maximum(deg, 1.0))
        vt = dinv_row * hwt_ref[...]
        vb_ref[...] = vt.astype(jnp.bfloat16)

        def body(j, acc):
            blk = cache_ref[pl.ds(j * BC, BC), :]
            vblk = vb_ref[:, pl.ds(j * BC, BC)]
            return acc + jax.lax.dot_general(
                vblk, blk, (((1,), (0,)), ((), ())),
                preferred_element_type=jnp.float32)

        acc = jax.lax.fori_loop(0, NC, body,
                                jnp.zeros((3, N), dtype=jnp.float32))
        out_t = dinv_row * (acc + miss_row * vt)
        out_ref[...] = jnp.transpose(out_t, (1, 0)) + bg_ref[...]


def kernel(x, adj, W1, b1, W3, b3, Wg, bg):
    b1r = b1.reshape(1, 16)
    b3r = b3.reshape(1, 3)
    bgr = bg.reshape(1, 3)
    out = pl.pallas_call(
        _gcn_kernel,
        grid=(NB,),
        in_specs=[
            pl.BlockSpec((N, 3), lambda i: (0, 0)),
            pl.BlockSpec((BR, N), lambda i: (i, 0)),
            pl.BlockSpec((BR, N), lambda i: (i + NB, 0)),
            pl.BlockSpec((3, 16), lambda i: (0, 0)),
            pl.BlockSpec((1, 16), lambda i: (0, 0)),
            pl.BlockSpec((16, 3), lambda i: (0, 0)),
            pl.BlockSpec((1, 3), lambda i: (0, 0)),
            pl.BlockSpec((3, 3), lambda i: (0, 0)),
            pl.BlockSpec((1, 3), lambda i: (0, 0)),
        ],
        out_specs=pl.BlockSpec((N, 3), lambda i: (0, 0)),
        out_shape=jax.ShapeDtypeStruct((N, 3), jnp.float32),
        scratch_shapes=[
            pltpu.VMEM((1, N), jnp.float32),
            pltpu.VMEM((1, N), jnp.float32),
            pltpu.VMEM((3, N), jnp.float32),
            pltpu.VMEM((3, N), jnp.bfloat16),
            pltpu.VMEM((N, N), jnp.bfloat16),
        ],
        compiler_params=pltpu.CompilerParams(
            dimension_semantics=("arbitrary",)),
    )(x, adj, adj, W1, b1r, W3, b3r, Wg, bgr)
    return out


# speedup vs baseline: 1.0383x; 1.0383x over previous
import jax
import jax.numpy as jnp
from jax.experimental import pallas as pl
from jax.experimental.pallas import tpu as pltpu

N = 4096
BR = 512
NB = N // BR
BC = 1024
NC = N // BC


def _gcn_kernel(x_ref, adj_ref, w1_ref, b1_ref, w3_ref,
                b3_ref, wg_ref, bg_ref, out_ref, deg_ref, miss_ref,
                hwt_ref, vb_ref, cache_ref):
    i = pl.program_id(0)

    @pl.when(i == 0)
    def _init():
        deg_ref[...] = jnp.zeros_like(deg_ref)
        h = jax.nn.relu(
            jnp.dot(x_ref[...], w1_ref[...],
                    preferred_element_type=jnp.float32) + b1_ref[...])
        h = jax.nn.relu(
            jnp.dot(h, w3_ref[...],
                    preferred_element_type=jnp.float32) + b3_ref[...])
        hw = jnp.dot(h, wg_ref[...], preferred_element_type=jnp.float32)
        hwt_ref[...] = jnp.transpose(hw, (1, 0))

    ones_row = jnp.ones((1, BR), dtype=jnp.bfloat16)
    r_ids = jax.lax.broadcasted_iota(jnp.int32, (BR, BR), 0)
    c_ids = jax.lax.broadcasted_iota(jnp.int32, (BR, BR), 1)
    eye = (r_ids == c_ids).astype(jnp.float32)

    def _consume(blk_ref, row0):
        blk = blk_ref[...].astype(jnp.bfloat16)
        cache_ref[pl.ds(row0, BR), :] = blk
        deg_ref[...] += jnp.dot(ones_row, blk,
                                preferred_element_type=jnp.float32)
        sq = blk_ref[:, pl.ds(row0, BR)]
        diag_row = jnp.sum(sq * eye, axis=0, keepdims=True)
        miss_ref[:, pl.ds(row0, BR)] = jnp.where(diag_row > 0, 0.0, 1.0)

    _consume(adj_ref, i * BR)

    @pl.when(i == NB - 1)
    def _finalize():
        miss_row = miss_ref[...]
        deg = deg_ref[...] + miss_row
        dinv_row = jax.lax.rsqrt(jnp.maximum(deg, 1.0))
        vt = dinv_row * hwt_ref[...]
        vb_ref[...] = vt.astype(jnp.bfloat16)

        def body(j, acc):
            blk = cache_ref[pl.ds(j * BC, BC), :]
            vblk = vb_ref[:, pl.ds(j * BC, BC)]
            return acc + jax.lax.dot_general(
                vblk, blk, (((1,), (0,)), ((), ())),
                preferred_element_type=jnp.float32)

        acc = jax.lax.fori_loop(0, NC, body,
                                jnp.zeros((3, N), dtype=jnp.float32))
        out_t = dinv_row * (acc + miss_row * vt)
        out_ref[...] = jnp.transpose(out_t, (1, 0)) + bg_ref[...]


def kernel(x, adj, W1, b1, W3, b3, Wg, bg):
    b1r = b1.reshape(1, 16)
    b3r = b3.reshape(1, 3)
    bgr = bg.reshape(1, 3)
    out = pl.pallas_call(
        _gcn_kernel,
        grid=(NB,),
        in_specs=[
            pl.BlockSpec((N, 3), lambda i: (0, 0)),
            pl.BlockSpec((BR, N), lambda i: (i, 0)),
            pl.BlockSpec((3, 16), lambda i: (0, 0)),
            pl.BlockSpec((1, 16), lambda i: (0, 0)),
            pl.BlockSpec((16, 3), lambda i: (0, 0)),
            pl.BlockSpec((1, 3), lambda i: (0, 0)),
            pl.BlockSpec((3, 3), lambda i: (0, 0)),
            pl.BlockSpec((1, 3), lambda i: (0, 0)),
        ],
        out_specs=pl.BlockSpec((N, 3), lambda i: (0, 0)),
        out_shape=jax.ShapeDtypeStruct((N, 3), jnp.float32),
        scratch_shapes=[
            pltpu.VMEM((1, N), jnp.float32),
            pltpu.VMEM((1, N), jnp.float32),
            pltpu.VMEM((3, N), jnp.float32),
            pltpu.VMEM((3, N), jnp.bfloat16),
            pltpu.VMEM((N, N), jnp.bfloat16),
        ],
        compiler_params=pltpu.CompilerParams(
            dimension_semantics=("arbitrary",)),
    )(x, adj, W1, b1r, W3, b3r, Wg, bgr)
    return out
